# Initial kernel scaffold; baseline (speedup 1.0000x reference)
import functools

import jax
import jax.numpy as jnp
from jax import lax
from jax.experimental import pallas as pl
from jax.experimental.pallas import tpu as pltpu
from jax.experimental.pallas import tpu_sc as plsc

N_NODES = 10000
D_FEAT = 256
NSEG = 64
NCLS = 10
NCLS_PAD = 16

ACC_W = 272
CH = 16
NCHUNKS = N_NODES // CH
NCORES = 2
NSUB = 16
NW = NCORES * NSUB
CPW = -(-NCHUNKS // NW)
LANES = 16

_mesh = plsc.VectorSubcoreMesh(core_axis_name="c", subcore_axis_name="s")


@functools.partial(
    pl.kernel,
    out_type=jax.ShapeDtypeStruct((NCORES, NSEG, ACC_W), jnp.float32),
    mesh=_mesh,
    scratch_types=[
        pltpu.VMEM((2, CH, D_FEAT), jnp.float32),
        pltpu.VMEM((2, CH), jnp.int32),
        pltpu.VMEM((NSEG, ACC_W), jnp.float32),
        pltpu.VMEM((NSEG,), jnp.int32),
        pltpu.VMEM_SHARED((NSEG, ACC_W), jnp.float32),
        pltpu.SemaphoreType.DMA,
        pltpu.SemaphoreType.DMA,
        pltpu.SemaphoreType.DMA,
        pltpu.SemaphoreType.DMA,
    ],
)
def _segsum_sc(x_hbm, batch_hbm, out_hbm, xbuf, bbuf, acc, idxv, shared,
               sx0, sx1, sb0, sb1):
    cid = lax.axis_index("c")
    sid = lax.axis_index("s")
    w = cid * NSUB + sid
    sx = (sx0, sx1)
    sb = (sb0, sb1)

    def _zrow(s, carry):
        for t in range(ACC_W // LANES):
            acc[s, pl.ds(t * LANES, LANES)] = jnp.zeros((LANES,), jnp.float32)
        return carry
    lax.fori_loop(0, NSEG, _zrow, 0)

    for t in range(NSEG // LANES):
        idxv[pl.ds(t * LANES, LANES)] = lax.iota(jnp.int32, 16) + t * LANES

    def _start(j, b):
        ck = w + NW * j

        @pl.when(ck < NCHUNKS)
        def _():
            pltpu.async_copy(x_hbm.at[pl.ds(ck * CH, CH)], xbuf.at[b], sx[b])
            pltpu.async_copy(batch_hbm.at[pl.ds(ck * CH, CH)], bbuf.at[b], sb[b])

    def _wait(b):
        pltpu.make_async_copy(x_hbm.at[pl.ds(0, CH)], xbuf.at[b], sx[b]).wait()
        pltpu.make_async_copy(batch_hbm.at[pl.ds(0, CH)], bbuf.at[b], sb[b]).wait()

    _start(0, 0)
    _start(1, 1)

    def _group(g, carry):
        for b in range(2):
            j = g * 2 + b
            ck = w + NW * j

            @pl.when(ck < NCHUNKS)
            def _():
                _wait(b)
                for i in range(CH):
                    seg = bbuf[b, i]
                    for t in range(D_FEAT // LANES):
                        plsc.addupdate(
                            acc.at[seg, pl.ds(t * LANES, LANES)],
                            xbuf[b, i, pl.ds(t * LANES, LANES)],
                        )
                    acc[seg, D_FEAT] = acc[seg, D_FEAT] + 1.0
            _start(j + 2, b)
        return carry
    lax.fori_loop(0, CPW // 2, _group, 0)

    plsc.subcore_barrier()

    @pl.when(sid == 0)
    def _():
        pltpu.sync_copy(acc, shared)

    plsc.subcore_barrier()

    @pl.when(sid != 0)
    def _():
        pltpu.sync_copy(acc, shared.at[idxv], add=True)

    plsc.subcore_barrier()

    @pl.when(sid == 0)
    def _():
        pltpu.sync_copy(shared, out_hbm.at[cid])


def _pool_linear_tc(p_ref, w_ref, b_ref, o_ref):
    p = p_ref[0] + p_ref[1]
    cnt = jnp.clip(p[:, D_FEAT:D_FEAT + 1], 1.0, None)
    pooled = p[:, :D_FEAT] / cnt
    o_ref[...] = lax.dot_general(
        pooled, w_ref[...], (((1,), (1,)), ((), ())),
        preferred_element_type=jnp.float32,
    ) + b_ref[...]


@jax.jit
def kernel(x, edge_index, batch, W, b):
    del edge_index
    partials = _segsum_sc(x, batch)
    w_pad = jnp.zeros((NCLS_PAD, D_FEAT), jnp.float32).at[:NCLS].set(W)
    b_pad = jnp.zeros((1, NCLS_PAD), jnp.float32).at[0, :NCLS].set(b)
    out = pl.pallas_call(
        _pool_linear_tc,
        out_shape=jax.ShapeDtypeStruct((NSEG, NCLS_PAD), jnp.float32),
    )(partials, w_pad, b_pad)
    return out[:, :NCLS]


# speedup vs baseline: 3.3214x; 3.3214x over previous
import functools

import jax
import jax.numpy as jnp
from jax import lax
from jax.experimental import pallas as pl
from jax.experimental.pallas import tpu as pltpu
from jax.experimental.pallas import tpu_sc as plsc

N_NODES = 10000
D_FEAT = 256
NSEG = 64
NCLS = 10
NCLS_PAD = 16

ACC_W = 272
CH = 16
NCHUNKS = N_NODES // CH
NCORES = 2
NSUB = 16
NW = NCORES * NSUB
CPW = -(-NCHUNKS // NW)
LANES = 16

_mesh = plsc.VectorSubcoreMesh(core_axis_name="c", subcore_axis_name="s")


@functools.partial(
    pl.kernel,
    out_type=jax.ShapeDtypeStruct((NW, NSEG, ACC_W), jnp.float32),
    mesh=_mesh,
    scratch_types=[
        pltpu.VMEM((2, CH, D_FEAT), jnp.float32),
        pltpu.VMEM((2, CH), jnp.int32),
        pltpu.VMEM((NSEG, ACC_W), jnp.float32),
        pltpu.SemaphoreType.DMA,
        pltpu.SemaphoreType.DMA,
        pltpu.SemaphoreType.DMA,
        pltpu.SemaphoreType.DMA,
    ],
)
def _segsum_sc(x_hbm, batch_hbm, out_hbm, xbuf, bbuf, acc, sx0, sx1, sb0, sb1):
    cid = lax.axis_index("c")
    sid = lax.axis_index("s")
    w = cid * NSUB + sid
    sx = (sx0, sx1)
    sb = (sb0, sb1)

    def _zrow(s, carry):
        for t in range(ACC_W // LANES):
            acc[s, pl.ds(t * LANES, LANES)] = jnp.zeros((LANES,), jnp.float32)
        return carry
    lax.fori_loop(0, NSEG, _zrow, 0)

    def _start(j, b):
        ck = w + NW * j

        @pl.when(ck < NCHUNKS)
        def _():
            pltpu.async_copy(x_hbm.at[pl.ds(ck * CH, CH)], xbuf.at[b], sx[b])
            pltpu.async_copy(batch_hbm.at[pl.ds(ck * CH, CH)], bbuf.at[b], sb[b])

    def _wait(b):
        pltpu.make_async_copy(x_hbm.at[pl.ds(0, CH)], xbuf.at[b], sx[b]).wait()
        pltpu.make_async_copy(batch_hbm.at[pl.ds(0, CH)], bbuf.at[b], sb[b]).wait()

    _start(0, 0)
    _start(1, 1)

    def _group(g, carry):
        for b in range(2):
            j = g * 2 + b
            ck = w + NW * j

            @pl.when(ck < NCHUNKS)
            def _():
                _wait(b)
                bvec = bbuf[b]
                e0 = jnp.clip(
                    1.0 - lax.iota(jnp.int32, 16).astype(jnp.float32), 0.0, 1.0)
                for i in range(CH):
                    seg = bvec[i]
                    for t in range(D_FEAT // LANES):
                        plsc.addupdate(
                            acc.at[seg, pl.ds(t * LANES, LANES)],
                            xbuf[b, i, pl.ds(t * LANES, LANES)],
                        )
                    plsc.addupdate(acc.at[seg, pl.ds(D_FEAT, LANES)], e0)
            _start(j + 2, b)
        return carry
    lax.fori_loop(0, CPW // 2, _group, 0)

    pltpu.sync_copy(acc, out_hbm.at[w])


def _pool_linear_tc(p_ref, w_ref, b_ref, o_ref):
    p = jnp.sum(p_ref[...], axis=0)
    cnt = jnp.clip(p[:, D_FEAT:D_FEAT + 1], 1.0, None)
    pooled = p[:, :D_FEAT] / cnt
    o_ref[...] = lax.dot_general(
        pooled, w_ref[...], (((1,), (1,)), ((), ())),
        preferred_element_type=jnp.float32,
    ) + b_ref[...]


@jax.jit
def kernel(x, edge_index, batch, W, b):
    del edge_index
    partials = _segsum_sc(x, batch)
    w_pad = jnp.zeros((NCLS_PAD, D_FEAT), jnp.float32).at[:NCLS].set(W)
    b_pad = jnp.zeros((1, NCLS_PAD), jnp.float32).at[0, :NCLS].set(b)
    out = pl.pallas_call(
        _pool_linear_tc,
        out_shape=jax.ShapeDtypeStruct((NSEG, NCLS_PAD), jnp.float32),
    )(partials, w_pad, b_pad)
    return out[:, :NCLS]
